# Initial kernel scaffold; baseline (speedup 1.0000x reference)
import jax
import jax.numpy as jnp
from jax.experimental import pallas as pl


def kernel(to_cache, kv_cache, slot_mapping):
    raise NotImplementedError("write your pallas kernel here")


# speedup vs baseline: 5.1261x; 5.1261x over previous
import jax
import jax.numpy as jnp
from jax import lax
from jax.experimental import pallas as pl
from jax.experimental.pallas import tpu as pltpu
from jax.experimental.pallas import tpu_sc as plsc

NUM_BLOCKS = 2048
BLOCK_SIZE = 16
NUM_HEADS = 8
HEAD_SIZE = 128
NUM_TOKENS = 16384

S = NUM_BLOCKS * BLOCK_SIZE
D = NUM_HEADS * HEAD_SIZE
NC = 2
NS = 16
L = 16
NW = NC * NS
SPW = S // NW
C = 64
TOK_BITS = 14
TOK_MASK = (1 << TOK_BITS) - 1
SENT = (1 << 31) - 1


def _body(tc_hbm, sm_hbm, out_ref, sm_v, w_v, sortbuf, slots_v, toks_v,
          gidx_v, sidx_v, rows_v, gsem, ssem):
    wid = lax.axis_index("c") * NS + lax.axis_index("s")
    base = wid * SPW
    iota = lax.iota(jnp.int32, L)

    pltpu.sync_copy(sm_hbm, sm_v)

    neg1 = jnp.full((L,), -1, jnp.int32)
    for i in range(SPW // L):
        w_v[pl.ds(i * L, L)] = neg1

    @pl.loop(0, NUM_TOKENS // L)
    def _scan(i):
        s = sm_v[pl.ds(i * L, L)]
        local = s - base
        valid = (local >= 0) & (local < SPW)
        tok = i * L + iota
        key = jnp.where(valid, (local << TOK_BITS) | tok, SENT)
        sorted_k, _ = plsc.sort_key_val(key, key)
        sortbuf[...] = sorted_k
        sk = sortbuf[...]
        nxt = plsc.load_gather(sortbuf, [jnp.minimum(iota + 1, L - 1)])
        keep = (sk != SENT) & (
            ((sk >> TOK_BITS) != (nxt >> TOK_BITS)) | (iota == L - 1))
        plsc.store_scatter(w_v, [sk >> TOK_BITS], sk & TOK_MASK, mask=keep)

    def _compact(i, cnt):
        w = w_v[pl.ds(i * L, L)]
        m = w >= 0
        mi = m.astype(jnp.int32)
        dest = cnt + plsc.cumsum(mi) - 1
        plsc.store_scatter(slots_v, [dest], base + i * L + iota, mask=m)
        plsc.store_scatter(toks_v, [dest], w, mask=m)
        return cnt + jnp.sum(mi, axis=0)

    cnt = lax.fori_loop(0, SPW // L, _compact, jnp.int32(0))

    lastv = jnp.full((L,), jnp.maximum(cnt - 1, 0), jnp.int32)
    pad_slot = plsc.load_gather(slots_v, [lastv])
    pad_tok = plsc.load_gather(toks_v, [lastv])
    for k in range(C // L):
        plsc.store_scatter(slots_v, [cnt + k * L + iota], pad_slot)
        plsc.store_scatter(toks_v, [cnt + k * L + iota], pad_tok)

    nchunks = (cnt + C - 1) // C

    @pl.loop(0, nchunks)
    def _chunk(j):
        for k in range(C // L):
            gidx_v[pl.ds(k * L, L)] = toks_v[pl.ds(j * C + k * L, L)]
            sidx_v[pl.ds(k * L, L)] = slots_v[pl.ds(j * C + k * L, L)]
        pltpu.async_copy(tc_hbm.at[gidx_v], rows_v, gsem).wait()
        pltpu.async_copy(rows_v, out_ref.at[sidx_v], ssem).wait()


def kernel(to_cache, kv_cache, slot_mapping):
    tc = to_cache.reshape(NUM_TOKENS, D)
    out_ref = jax.new_ref(kv_cache.reshape(S, D))
    scatter = pl.kernel(
        _body,
        out_type=(),
        mesh=plsc.VectorSubcoreMesh(
            core_axis_name="c", subcore_axis_name="s",
            num_cores=NC, num_subcores=NS),
        compiler_params=pltpu.CompilerParams(needs_layout_passes=False),
        scratch_types=[
            pltpu.VMEM((NUM_TOKENS,), jnp.int32),
            pltpu.VMEM((SPW,), jnp.int32),
            pltpu.VMEM((L,), jnp.int32),
            pltpu.VMEM((SPW + C,), jnp.int32),
            pltpu.VMEM((SPW + C,), jnp.int32),
            pltpu.VMEM((C,), jnp.int32),
            pltpu.VMEM((C,), jnp.int32),
            pltpu.VMEM((C, D), jnp.float32),
            pltpu.SemaphoreType.DMA,
            pltpu.SemaphoreType.DMA,
        ],
    )
    scatter(tc, slot_mapping, out_ref)
    return jax.freeze(out_ref).reshape(
        NUM_BLOCKS, BLOCK_SIZE, NUM_HEADS, HEAD_SIZE)
